# baseline (speedup 1.0000x reference)
import functools

import jax
import jax.numpy as jnp
from jax import lax
from jax.experimental import pallas as pl
from jax.experimental.pallas import tpu as pltpu
from jax.experimental.pallas import tpu_sc as plsc

NUM_ITEM = 8000
NUM_USER = 2000
NN = NUM_ITEM + NUM_USER
D = 128
E = 320000

NUM_CORES = 2
NUM_SUBCORES = 16
NW = NUM_CORES * NUM_SUBCORES
BLK = 64
NBLK = 158
EPW = NBLK * BLK
E_PAD = EPW * NW
N_PAD = 10112
XW_PAD = 10016
ROWS_PER_TILE = N_PAD // NUM_SUBCORES


def _leaky(v):
    return jnp.where(v >= 0, v, 0.01 * v)


def _pre_body(feat_ref, ufeat_ref, wq_ref, bq_ref, convw_ref, wlin_ref,
              blin_ref, idemb_ref, xw_ref, xhat_ref):
    uf = jnp.tanh(
        jnp.dot(ufeat_ref[...], wq_ref[...],
                preferred_element_type=jnp.float32) + bq_ref[...])
    x = jnp.concatenate([feat_ref[...], uf], axis=0)
    n = jnp.sqrt(jnp.sum(x * x, axis=-1, keepdims=True))
    x = x / jnp.maximum(n, 1e-12)
    xw_ref[...] = jnp.dot(x, convw_ref[...], preferred_element_type=jnp.float32)
    xhat_ref[...] = _leaky(
        jnp.dot(x, wlin_ref[...], preferred_element_type=jnp.float32)
        + blin_ref[...]) + idemb_ref[...]


def _sc_edge_body(xw_hbm, src_hbm, dst_hbm, zero_hbm, out_hbm,
                  u_sh, src_idx, dst_idx, sdst_idx, srows, drows, orows,
                  sem_s0, sem_s1, sem_d0, sem_d1, sem_u0, sem_u1,
                  sem_is0, sem_is1, sem_id0, sem_id1):
    c = lax.axis_index("c")
    s = lax.axis_index("s")
    wid = s * NUM_CORES + c
    sem_s = (sem_s0, sem_s1)
    sem_d = (sem_d0, sem_d1)
    sem_u = (sem_u0, sem_u1)
    sem_is = (sem_is0, sem_is1)
    sem_id = (sem_id0, sem_id1)

    pltpu.sync_copy(zero_hbm.at[pl.ds(s * ROWS_PER_TILE, ROWS_PER_TILE)],
                    u_sh.at[pl.ds(s * ROWS_PER_TILE, ROWS_PER_TILE)])
    plsc.subcore_barrier()

    lane = lax.iota(jnp.int32, 16)

    def idx_fetch(b, par):
        base = wid * EPW + b * BLK
        pltpu.async_copy(src_hbm.at[pl.ds(base, BLK)], src_idx.at[par],
                         sem_is[par])
        pltpu.async_copy(dst_hbm.at[pl.ds(base, BLK)], dst_idx.at[par],
                         sem_id[par])

    def wait_idx(b, par):
        base = wid * EPW + b * BLK
        pltpu.make_async_copy(src_hbm.at[pl.ds(base, BLK)], src_idx.at[par],
                              sem_is[par]).wait()
        pltpu.make_async_copy(dst_hbm.at[pl.ds(base, BLK)], dst_idx.at[par],
                              sem_id[par]).wait()

    def gather_issue(par):
        pltpu.async_copy(xw_hbm.at[src_idx.at[par]], srows.at[par], sem_s[par])
        pltpu.async_copy(xw_hbm.at[dst_idx.at[par]], drows.at[par], sem_d[par])

    def wait_rows(par):
        pltpu.make_async_copy(xw_hbm.at[src_idx.at[par]], srows.at[par],
                              sem_s[par]).wait()
        pltpu.make_async_copy(xw_hbm.at[dst_idx.at[par]], drows.at[par],
                              sem_d[par]).wait()

    def drain_scatter(par):
        pltpu.make_async_copy(orows.at[par], u_sh.at[sdst_idx.at[par]],
                              sem_u[par]).wait()

    def compute(par):
        sref = srows.at[par]
        dref = drows.at[par]
        oref = orows.at[par]

        def group_body(g, carry2):
            erow = g * 16 + lane

            def dot_chunk(dd, acc):
                for d2 in range(16):
                    dcol = jnp.full((16,), dd * 16 + d2, jnp.int32)
                    sv = plsc.load_gather(sref, [erow, dcol])
                    dv = plsc.load_gather(dref, [erow, dcol])
                    acc = acc + sv * dv
                return acc

            z = lax.fori_loop(0, 8, dot_chunk, jnp.zeros((16,), jnp.float32))
            w = jnp.exp(z / (1.0 + jnp.exp(-z)))

            def scale_chunk(dd, carry3):
                for d2 in range(16):
                    dcol = jnp.full((16,), dd * 16 + d2, jnp.int32)
                    sv = plsc.load_gather(sref, [erow, dcol])
                    plsc.store_scatter(oref, [erow, dcol], sv * w)
                return carry3

            lax.fori_loop(0, 8, scale_chunk, 0)
            return carry2

        lax.fori_loop(0, BLK // 16, group_body, 0)

    def step(b, par, is_ge2, fetch_next):
        wait_rows(par)
        pl.when(is_ge2)(lambda: drain_scatter(par))
        for k in range(BLK // 16):
            sdst_idx[par, pl.ds(k * 16, 16)] = dst_idx[par, pl.ds(k * 16, 16)]
        pl.when(fetch_next)(lambda: idx_fetch(b + 2, par))
        compute(par)
        pltpu.async_copy(orows.at[par], u_sh.at[sdst_idx.at[par]],
                         sem_u[par], add=True)

    idx_fetch(0, 0)
    wait_idx(0, 0)
    gather_issue(0)
    idx_fetch(1, 1)

    def pair_body(p, carry):
        b0 = 2 * p
        wait_idx(b0 + 1, 1)
        gather_issue(1)
        step(b0, 0, p > 0, b0 + 2 < NBLK)

        def next_gather():
            wait_idx(b0 + 2, 0)
            gather_issue(0)

        pl.when(b0 + 2 < NBLK)(next_gather)
        step(b0 + 1, 1, p > 0, b0 + 3 < NBLK)
        return carry

    lax.fori_loop(0, NBLK // 2, pair_body, 0)
    drain_scatter(0)
    drain_scatter(1)

    plsc.subcore_barrier()
    pltpu.sync_copy(u_sh.at[pl.ds(s * ROWS_PER_TILE, ROWS_PER_TILE)],
                    out_hbm.at[c, pl.ds(s * ROWS_PER_TILE, ROWS_PER_TILE)])


_sc_edge = functools.partial(
    pl.kernel,
    out_type=jax.ShapeDtypeStruct((NUM_CORES, N_PAD, D), jnp.float32),
    mesh=plsc.VectorSubcoreMesh(core_axis_name="c", subcore_axis_name="s"),
    scratch_types=[
        pltpu.VMEM_SHARED((N_PAD, D), jnp.float32),
        pltpu.VMEM((2, BLK), jnp.int32),
        pltpu.VMEM((2, BLK), jnp.int32),
        pltpu.VMEM((2, BLK), jnp.int32),
        pltpu.VMEM((2, BLK, D), jnp.float32),
        pltpu.VMEM((2, BLK, D), jnp.float32),
        pltpu.VMEM((2, BLK, D), jnp.float32),
        pltpu.SemaphoreType.DMA,
        pltpu.SemaphoreType.DMA,
        pltpu.SemaphoreType.DMA,
        pltpu.SemaphoreType.DMA,
        pltpu.SemaphoreType.DMA,
        pltpu.SemaphoreType.DMA,
        pltpu.SemaphoreType.DMA,
        pltpu.SemaphoreType.DMA,
        pltpu.SemaphoreType.DMA,
        pltpu.SemaphoreType.DMA,
    ],
    compiler_params=pltpu.CompilerParams(needs_layout_passes=False),
)(_sc_edge_body)


def _post_body(u_ref, xhat_ref, wg_ref, bg_ref, out_ref):
    u = u_ref[0, :NN, :] + u_ref[1, :NN, :]
    n = jnp.sqrt(jnp.sum(u * u, axis=-1, keepdims=True))
    h = _leaky(u / jnp.maximum(n, 1e-12))
    out_ref[...] = _leaky(
        jnp.dot(h, wg_ref[...], preferred_element_type=jnp.float32)
        + bg_ref[...] + xhat_ref[...])


@jax.jit
def kernel(features, user_features, id_embedding, W_user, b_user, conv_w,
           W_lin1, b_lin1, W_g1, b_g1, edge_index):
    xw, x_hat = pl.pallas_call(
        _pre_body,
        out_shape=[
            jax.ShapeDtypeStruct((NN, D), jnp.float32),
            jax.ShapeDtypeStruct((NN, 64), jnp.float32),
        ],
    )(features, user_features, W_user, b_user, conv_w, W_lin1, b_lin1,
      id_embedding)

    ei = edge_index.astype(jnp.int32)
    pad_src = jnp.zeros((E_PAD - E,), jnp.int32)
    pad_dst = jnp.full((E_PAD - E,), NN, jnp.int32)
    srcp = jnp.concatenate([ei[0], pad_src])
    dstp = jnp.concatenate([ei[1], pad_dst])
    xw_pad = jnp.concatenate(
        [xw, jnp.zeros((XW_PAD - NN, D), jnp.float32)], axis=0)
    zeros_hbm = jnp.zeros((N_PAD, D), jnp.float32)

    u2 = _sc_edge(xw_pad, srcp, dstp, zeros_hbm)

    out = pl.pallas_call(
        _post_body,
        out_shape=jax.ShapeDtypeStruct((NN, 64), jnp.float32),
    )(u2, x_hat, W_g1, b_g1)
    return out


# speedup vs baseline: 6.1936x; 6.1936x over previous
import functools

import jax
import jax.numpy as jnp
from jax import lax
from jax.experimental import pallas as pl
from jax.experimental.pallas import tpu as pltpu
from jax.experimental.pallas import tpu_sc as plsc

NUM_ITEM = 8000
NUM_USER = 2000
NN = NUM_ITEM + NUM_USER
D = 128
E = 320000

NUM_CORES = 2
NUM_SUBCORES = 16
NW = NUM_CORES * NUM_SUBCORES
BLK = 64
NBLK = 158
EPW = NBLK * BLK
E_PAD = EPW * NW
N_PAD = 10112
XW_PAD = 10016
ROWS_PER_TILE = N_PAD // NUM_SUBCORES


def _leaky(v):
    return jnp.where(v >= 0, v, 0.01 * v)


def _pre_body(feat_ref, ufeat_ref, wq_ref, bq_ref, convw_ref, wlin_ref,
              blin_ref, idemb_ref, xw_ref, xhat_ref):
    uf = jnp.tanh(
        jnp.dot(ufeat_ref[...], wq_ref[...],
                preferred_element_type=jnp.float32) + bq_ref[...])
    x = jnp.concatenate([feat_ref[...], uf], axis=0)
    n = jnp.sqrt(jnp.sum(x * x, axis=-1, keepdims=True))
    x = x / jnp.maximum(n, 1e-12)
    xw_ref[...] = jnp.dot(x, convw_ref[...], preferred_element_type=jnp.float32)
    xhat_ref[...] = _leaky(
        jnp.dot(x, wlin_ref[...], preferred_element_type=jnp.float32)
        + blin_ref[...]) + idemb_ref[...]


def _sc_edge_body(xw_hbm, src_hbm, dst_hbm, zero_hbm, out_hbm,
                  u_sh, src_idx, dst_idx, sdst_idx, srows, drows, orows,
                  sem_s0, sem_s1, sem_d0, sem_d1, sem_u0, sem_u1,
                  sem_is0, sem_is1, sem_id0, sem_id1):
    c = lax.axis_index("c")
    s = lax.axis_index("s")
    wid = s * NUM_CORES + c
    sem_s = (sem_s0, sem_s1)
    sem_d = (sem_d0, sem_d1)
    sem_u = (sem_u0, sem_u1)
    sem_is = (sem_is0, sem_is1)
    sem_id = (sem_id0, sem_id1)

    pltpu.sync_copy(zero_hbm.at[pl.ds(s * ROWS_PER_TILE, ROWS_PER_TILE)],
                    u_sh.at[pl.ds(s * ROWS_PER_TILE, ROWS_PER_TILE)])
    plsc.subcore_barrier()

    lane = lax.iota(jnp.int32, 16)

    def idx_fetch(b, par):
        base = wid * EPW + b * BLK
        pltpu.async_copy(src_hbm.at[pl.ds(base, BLK)], src_idx.at[par],
                         sem_is[par])
        pltpu.async_copy(dst_hbm.at[pl.ds(base, BLK)], dst_idx.at[par],
                         sem_id[par])

    def wait_idx(b, par):
        base = wid * EPW + b * BLK
        pltpu.make_async_copy(src_hbm.at[pl.ds(base, BLK)], src_idx.at[par],
                              sem_is[par]).wait()
        pltpu.make_async_copy(dst_hbm.at[pl.ds(base, BLK)], dst_idx.at[par],
                              sem_id[par]).wait()

    def gather_issue(par):
        pltpu.async_copy(xw_hbm.at[src_idx.at[par]], srows.at[par], sem_s[par])
        pltpu.async_copy(xw_hbm.at[dst_idx.at[par]], drows.at[par], sem_d[par])

    def wait_rows(par):
        pltpu.make_async_copy(xw_hbm.at[src_idx.at[par]], srows.at[par],
                              sem_s[par]).wait()
        pltpu.make_async_copy(xw_hbm.at[dst_idx.at[par]], drows.at[par],
                              sem_d[par]).wait()

    def drain_scatter(par):
        pltpu.make_async_copy(orows.at[par], u_sh.at[sdst_idx.at[par]],
                              sem_u[par]).wait()

    def compute(par):
        sref = srows.at[par]
        dref = drows.at[par]
        oref = orows.at[par]

        def dot_z(e):
            ef = jnp.full((16,), e, jnp.int32)
            svs = []
            acc = jnp.zeros((16,), jnp.float32)
            for k in range(8):
                col = k * 16 + lane
                sv = plsc.load_gather(sref, [ef, col])
                dv = plsc.load_gather(dref, [ef, col])
                svs.append(sv)
                acc = acc + sv * dv
            zv = jnp.full((16,), jnp.sum(acc), jnp.float32)
            bc = jnp.exp(zv / (1.0 + jnp.exp(-zv)))
            return tuple(svs), bc

        def scale(e, svs, bc):
            ef = jnp.full((16,), e, jnp.int32)
            for k in range(8):
                plsc.store_scatter(oref, [ef, k * 16 + lane], svs[k] * bc)

        svs0, bc0 = dot_z(0)

        def body(i, carry):
            svs_p, bc_p = carry
            svs_i, bc_i = dot_z(i)
            scale(i - 1, svs_p, bc_p)
            return (svs_i, bc_i)

        svs_l, bc_l = lax.fori_loop(1, BLK, body, (svs0, bc0))
        scale(BLK - 1, svs_l, bc_l)

    def step(b, par, is_ge2, fetch_next):
        wait_rows(par)
        pl.when(is_ge2)(lambda: drain_scatter(par))
        for k in range(BLK // 16):
            sdst_idx[par, pl.ds(k * 16, 16)] = dst_idx[par, pl.ds(k * 16, 16)]
        pl.when(fetch_next)(lambda: idx_fetch(b + 2, par))
        compute(par)
        pltpu.async_copy(orows.at[par], u_sh.at[sdst_idx.at[par]],
                         sem_u[par], add=True)

    idx_fetch(0, 0)
    wait_idx(0, 0)
    gather_issue(0)
    idx_fetch(1, 1)

    def pair_body(p, carry):
        b0 = 2 * p
        wait_idx(b0 + 1, 1)
        gather_issue(1)
        step(b0, 0, p > 0, b0 + 2 < NBLK)

        def next_gather():
            wait_idx(b0 + 2, 0)
            gather_issue(0)

        pl.when(b0 + 2 < NBLK)(next_gather)
        step(b0 + 1, 1, p > 0, b0 + 3 < NBLK)
        return carry

    lax.fori_loop(0, NBLK // 2, pair_body, 0)
    drain_scatter(0)
    drain_scatter(1)

    plsc.subcore_barrier()
    pltpu.sync_copy(u_sh.at[pl.ds(s * ROWS_PER_TILE, ROWS_PER_TILE)],
                    out_hbm.at[c, pl.ds(s * ROWS_PER_TILE, ROWS_PER_TILE)])


_sc_edge = functools.partial(
    pl.kernel,
    out_type=jax.ShapeDtypeStruct((NUM_CORES, N_PAD, D), jnp.float32),
    mesh=plsc.VectorSubcoreMesh(core_axis_name="c", subcore_axis_name="s"),
    scratch_types=[
        pltpu.VMEM_SHARED((N_PAD, D), jnp.float32),
        pltpu.VMEM((2, BLK), jnp.int32),
        pltpu.VMEM((2, BLK), jnp.int32),
        pltpu.VMEM((2, BLK), jnp.int32),
        pltpu.VMEM((2, BLK, D), jnp.float32),
        pltpu.VMEM((2, BLK, D), jnp.float32),
        pltpu.VMEM((2, BLK, D), jnp.float32),
        pltpu.SemaphoreType.DMA,
        pltpu.SemaphoreType.DMA,
        pltpu.SemaphoreType.DMA,
        pltpu.SemaphoreType.DMA,
        pltpu.SemaphoreType.DMA,
        pltpu.SemaphoreType.DMA,
        pltpu.SemaphoreType.DMA,
        pltpu.SemaphoreType.DMA,
        pltpu.SemaphoreType.DMA,
        pltpu.SemaphoreType.DMA,
    ],
    compiler_params=pltpu.CompilerParams(needs_layout_passes=False),
)(_sc_edge_body)


def _post_body(u_ref, xhat_ref, wg_ref, bg_ref, out_ref):
    u = u_ref[0, :NN, :] + u_ref[1, :NN, :]
    n = jnp.sqrt(jnp.sum(u * u, axis=-1, keepdims=True))
    h = _leaky(u / jnp.maximum(n, 1e-12))
    out_ref[...] = _leaky(
        jnp.dot(h, wg_ref[...], preferred_element_type=jnp.float32)
        + bg_ref[...] + xhat_ref[...])


@jax.jit
def kernel(features, user_features, id_embedding, W_user, b_user, conv_w,
           W_lin1, b_lin1, W_g1, b_g1, edge_index):
    xw, x_hat = pl.pallas_call(
        _pre_body,
        out_shape=[
            jax.ShapeDtypeStruct((NN, D), jnp.float32),
            jax.ShapeDtypeStruct((NN, 64), jnp.float32),
        ],
    )(features, user_features, W_user, b_user, conv_w, W_lin1, b_lin1,
      id_embedding)

    ei = edge_index.astype(jnp.int32)
    pad_src = jnp.zeros((E_PAD - E,), jnp.int32)
    pad_dst = jnp.full((E_PAD - E,), NN, jnp.int32)
    srcp = jnp.concatenate([ei[0], pad_src])
    dstp = jnp.concatenate([ei[1], pad_dst])
    xw_pad = jnp.concatenate(
        [xw, jnp.zeros((XW_PAD - NN, D), jnp.float32)], axis=0)
    zeros_hbm = jnp.zeros((N_PAD, D), jnp.float32)

    u2 = _sc_edge(xw_pad, srcp, dstp, zeros_hbm)

    out = pl.pallas_call(
        _post_body,
        out_shape=jax.ShapeDtypeStruct((NN, 64), jnp.float32),
    )(u2, x_hat, W_g1, b_g1)
    return out
